# baseline (speedup 1.0000x reference)
import functools

import jax
import jax.numpy as jnp
from jax import lax
from jax.experimental import pallas as pl
from jax.experimental.pallas import tpu as pltpu
from jax.experimental.pallas import tpu_sc as plsc

N_SPEAKERS = 100000
EMBED_DIM = 64
PAD_DIM = 128
BATCH = 16384

_NC = 2
_NS = 16
_NW = _NC * _NS
_BPW = BATCH // _NW
_CH = 128
_NCH = _BPW // _CH

_TB = 1024
_GRID = -(-N_SPEAKERS // _TB)

_mesh = plsc.VectorSubcoreMesh(core_axis_name="c", subcore_axis_name="s")


@functools.partial(
    pl.pallas_call,
    grid=(_GRID,),
    in_specs=[pl.BlockSpec((EMBED_DIM, _TB), lambda i: (0, i))],
    out_specs=pl.BlockSpec((_TB, PAD_DIM), lambda i: (i, 0)),
    out_shape=jax.ShapeDtypeStruct((N_SPEAKERS, PAD_DIM), jnp.float32),
)
def _transpose_pad(tt_ref, out_ref):
    blk = tt_ref[...]
    eye = jnp.eye(EMBED_DIM, dtype=jnp.float32)
    out_ref[:, :EMBED_DIM] = lax.dot_general(
        blk, eye, (((0,), (0,)), ((), ())),
        preferred_element_type=jnp.float32,
    )
    out_ref[:, EMBED_DIM:] = jnp.zeros((_TB, PAD_DIM - EMBED_DIM), jnp.float32)


@functools.partial(
    pl.kernel,
    mesh=_mesh,
    out_type=jax.ShapeDtypeStruct((BATCH, PAD_DIM), jnp.float32),
    scratch_types=[
        pltpu.VMEM((_BPW,), jnp.int32),
        pltpu.VMEM((_BPW, PAD_DIM), jnp.float32),
    ] + [pltpu.SemaphoreType.DMA] * _NCH,
)
def _lookup(speaker_hbm, table_hbm, out_hbm, idx_v, rows_v, *sems):
    wid = lax.axis_index("s") * _NC + lax.axis_index("c")
    base = wid * _BPW
    pltpu.sync_copy(speaker_hbm.at[pl.ds(base, _BPW)], idx_v)
    copies = []
    for k in range(_NCH):
        copies.append(
            pltpu.async_copy(
                table_hbm.at[idx_v.at[pl.ds(k * _CH, _CH)]],
                rows_v.at[pl.ds(k * _CH, _CH)],
                sems[k],
            )
        )
    for c in copies:
        c.wait()
    pltpu.sync_copy(rows_v, out_hbm.at[pl.ds(base, _BPW)])


def kernel(speaker, table):
    table_p = _transpose_pad(table.T)
    out_p = _lookup(speaker.astype(jnp.int32), table_p)
    return out_p[:, :EMBED_DIM]


# speedup vs baseline: 1.5829x; 1.5829x over previous
import functools

import jax
import jax.numpy as jnp
from jax import lax
from jax.experimental import pallas as pl
from jax.experimental.pallas import tpu as pltpu
from jax.experimental.pallas import tpu_sc as plsc

N_SPEAKERS = 100000
EMBED_DIM = 64
PAD_DIM = 128
BATCH = 16384

_NC = 2
_NS = 16
_NW = _NC * _NS
_BPW = BATCH // _NW
_CH = 128
_NCH = _BPW // _CH

_TB = 4096
_GRID = -(-N_SPEAKERS // _TB)

_mesh = plsc.VectorSubcoreMesh(core_axis_name="c", subcore_axis_name="s")


@functools.partial(
    pl.pallas_call,
    grid=(_GRID,),
    in_specs=[pl.BlockSpec((EMBED_DIM, _TB), lambda i: (0, i))],
    out_specs=pl.BlockSpec((_TB, PAD_DIM), lambda i: (i, 0)),
    out_shape=jax.ShapeDtypeStruct((N_SPEAKERS, PAD_DIM), jnp.float32),
)
def _transpose_pad(tt_ref, out_ref):
    blk = tt_ref[...]
    eye = jnp.eye(EMBED_DIM, dtype=jnp.float32)
    out_ref[:, :EMBED_DIM] = lax.dot_general(
        blk, eye, (((0,), (0,)), ((), ())),
        preferred_element_type=jnp.float32,
    )
    out_ref[:, EMBED_DIM:] = jnp.zeros((_TB, PAD_DIM - EMBED_DIM), jnp.float32)


@functools.partial(
    pl.kernel,
    mesh=_mesh,
    out_type=jax.ShapeDtypeStruct((BATCH, PAD_DIM), jnp.float32),
    scratch_types=[
        pltpu.VMEM((_BPW,), jnp.int32),
        pltpu.VMEM((_BPW, PAD_DIM), jnp.float32),
    ] + [pltpu.SemaphoreType.DMA] * _NCH,
)
def _lookup(speaker_hbm, table_hbm, out_hbm, idx_v, rows_v, *sems):
    wid = lax.axis_index("s") * _NC + lax.axis_index("c")
    base = wid * _BPW
    pltpu.sync_copy(speaker_hbm.at[pl.ds(base, _BPW)], idx_v)
    copies = []
    for k in range(_NCH):
        copies.append(
            pltpu.async_copy(
                table_hbm.at[idx_v.at[pl.ds(k * _CH, _CH)]],
                rows_v.at[pl.ds(k * _CH, _CH)],
                sems[k],
            )
        )
    for c in copies:
        c.wait()
    pltpu.sync_copy(rows_v, out_hbm.at[pl.ds(base, _BPW)])


def kernel(speaker, table):
    table_p = _transpose_pad(table.T)
    out_p = _lookup(speaker.astype(jnp.int32), table_p)
    return out_p[:, :EMBED_DIM]


# speedup vs baseline: 1.7871x; 1.1290x over previous
import functools

import jax
import jax.numpy as jnp
from jax import lax
from jax.experimental import pallas as pl
from jax.experimental.pallas import tpu as pltpu
from jax.experimental.pallas import tpu_sc as plsc

N_SPEAKERS = 100000
EMBED_DIM = 64
PAD_DIM = 128
BATCH = 16384

_NC = 2
_NS = 16
_NW = _NC * _NS
_BPW = BATCH // _NW
_CH = 128
_NCH = _BPW // _CH

_TB = 8192
_GRID = -(-N_SPEAKERS // _TB)

_mesh = plsc.VectorSubcoreMesh(core_axis_name="c", subcore_axis_name="s")


@functools.partial(
    pl.pallas_call,
    grid=(_GRID,),
    in_specs=[pl.BlockSpec((EMBED_DIM, _TB), lambda i: (0, i))],
    out_specs=pl.BlockSpec((_TB, PAD_DIM), lambda i: (i, 0)),
    out_shape=jax.ShapeDtypeStruct((N_SPEAKERS, PAD_DIM), jnp.float32),
)
def _transpose_pad(tt_ref, out_ref):
    blk = tt_ref[...]
    eye = jnp.eye(EMBED_DIM, dtype=jnp.float32)
    out_ref[:, :EMBED_DIM] = lax.dot_general(
        blk, eye, (((0,), (0,)), ((), ())),
        preferred_element_type=jnp.float32,
    )
    out_ref[:, EMBED_DIM:] = jnp.zeros((_TB, PAD_DIM - EMBED_DIM), jnp.float32)


@functools.partial(
    pl.kernel,
    mesh=_mesh,
    out_type=jax.ShapeDtypeStruct((BATCH, PAD_DIM), jnp.float32),
    scratch_types=[
        pltpu.VMEM((_BPW,), jnp.int32),
        pltpu.VMEM((_BPW, PAD_DIM), jnp.float32),
    ] + [pltpu.SemaphoreType.DMA] * _NCH,
)
def _lookup(speaker_hbm, table_hbm, out_hbm, idx_v, rows_v, *sems):
    wid = lax.axis_index("s") * _NC + lax.axis_index("c")
    base = wid * _BPW
    pltpu.sync_copy(speaker_hbm.at[pl.ds(base, _BPW)], idx_v)
    copies = []
    for k in range(_NCH):
        copies.append(
            pltpu.async_copy(
                table_hbm.at[idx_v.at[pl.ds(k * _CH, _CH)]],
                rows_v.at[pl.ds(k * _CH, _CH)],
                sems[k],
            )
        )
    for c in copies:
        c.wait()
    pltpu.sync_copy(rows_v, out_hbm.at[pl.ds(base, _BPW)])


def kernel(speaker, table):
    table_p = _transpose_pad(table.T)
    out_p = _lookup(speaker.astype(jnp.int32), table_p)
    return out_p[:, :EMBED_DIM]
